# baseline (speedup 1.0000x reference)
import functools

import jax
import jax.numpy as jnp
from jax import lax
from jax.experimental import pallas as pl
from jax.experimental.pallas import tpu as pltpu
from jax.experimental.pallas import tpu_sc as plsc

NC = 2
NS = 16
NW = NC * NS
CHUNK = 80
NB = 4


def _sc_segment_sum(x, src, dst, *, n_pad, d, epw):
    mesh = plsc.VectorSubcoreMesh(core_axis_name="c", subcore_axis_name="s")
    rows_per_sub = n_pad // NS
    n_chunks = epw // CHUNK
    assert epw == n_chunks * CHUNK and n_chunks > 2 * NB
    n_loop = (n_chunks - 1) // NB * NB

    @functools.partial(
        pl.kernel,
        out_type=jax.ShapeDtypeStruct((NC, n_pad, d), jnp.float32),
        mesh=mesh,
        scratch_types=(
            [pltpu.VMEM((CHUNK, d), jnp.float32)] * NB
            + [pltpu.VMEM((CHUNK,), jnp.int32)] * NB
            + [pltpu.VMEM((CHUNK,), jnp.int32)] * NB
            + [pltpu.VMEM_SHARED((n_pad, d), jnp.float32)]
            + [pltpu.SemaphoreType.DMA] * (4 * NB)
        ),
    )
    def k(x_hbm, src_hbm, dst_hbm, out_hbm, *refs):
        buf = refs[0:NB]
        srcb = refs[NB:2 * NB]
        dstb = refs[2 * NB:3 * NB]
        acc = refs[3 * NB]
        sg = refs[3 * NB + 1:4 * NB + 1]
        ss = refs[4 * NB + 1:5 * NB + 1]
        ssrc = refs[5 * NB + 1:6 * NB + 1]
        sd = refs[6 * NB + 1:7 * NB + 1]

        c = lax.axis_index("c")
        s = lax.axis_index("s")
        wid = c * NS + s

        ebase = wid * epw

        def src_stage(m, q):
            pltpu.async_copy(src_hbm.at[pl.ds(ebase + m * CHUNK, CHUNK)],
                             srcb[q], ssrc[q])

        def src_wait(m, q):
            pltpu.make_async_copy(src_hbm.at[pl.ds(ebase + m * CHUNK, CHUNK)],
                                  srcb[q], ssrc[q]).wait()

        def dst_stage(m, q):
            pltpu.async_copy(dst_hbm.at[pl.ds(ebase + m * CHUNK, CHUNK)],
                             dstb[q], sd[q])

        def dst_wait(m, q):
            pltpu.make_async_copy(dst_hbm.at[pl.ds(ebase + m * CHUNK, CHUNK)],
                                  dstb[q], sd[q]).wait()

        def gather(q):
            pltpu.async_copy(x_hbm.at[srcb[q]], buf[q], sg[q])

        def gather_wait(q):
            pltpu.make_async_copy(x_hbm.at[srcb[q]], buf[q], sg[q]).wait()

        def scat(q):
            pltpu.async_copy(buf[q], acc.at[dstb[q]], ss[q], add=True)

        def scat_wait(q):
            pltpu.make_async_copy(buf[q], acc.at[dstb[q]], ss[q]).wait()

        for q in range(NB):
            src_stage(q, q)
        dst_stage(0, 0)
        dst_stage(1, 1)

        zv = jnp.zeros((16,), jnp.float32)
        zb = buf[NB - 1]

        @pl.loop(0, CHUNK)
        def _(r):
            @pl.loop(0, d, step=16)
            def _(q):
                zb.at[r][pl.ds(q, 16)] = zv

        base = s * rows_per_sub
        nfull_z = rows_per_sub // CHUNK
        rem_z = rows_per_sub - nfull_z * CHUNK

        @pl.loop(0, nfull_z)
        def _(t):
            pltpu.sync_copy(zb, acc.at[pl.ds(base + t * CHUNK, CHUNK)])

        if rem_z:
            pltpu.sync_copy(
                zb.at[pl.ds(0, rem_z)],
                acc.at[pl.ds(base + nfull_z * CHUNK, rem_z)],
            )

        src_wait(0, 0)
        gather(0)
        src_wait(1, 1)
        gather(1)
        plsc.subcore_barrier()

        @pl.loop(0, n_loop, step=NB)
        def _(j):
            for q in range(NB):
                m = j + q
                gather_wait(q)

                @pl.when(m + NB < n_chunks)
                def _():
                    src_stage(m + NB, q)

                dst_wait(m, q)
                scat(q)

                @pl.when(m >= 2)
                def _():
                    scat_wait((q - 2) % NB)

                @pl.when(m + 2 < n_chunks)
                def _():
                    qn = (q + 2) % NB
                    dst_stage(m + 2, qn)
                    src_wait(m + 2, qn)
                    gather(qn)

        for m in range(n_loop, n_chunks):
            q = m % NB
            gather_wait(q)
            dst_wait(m, q)
            scat(q)
            scat_wait((q - 2) % NB)

        scat_wait((n_chunks - 2) % NB)
        scat_wait((n_chunks - 1) % NB)

        plsc.subcore_barrier()
        pltpu.sync_copy(
            acc.at[pl.ds(base, rows_per_sub)],
            out_hbm.at[c, pl.ds(base, rows_per_sub)],
        )

    return k(x, src, dst)


def _combine(x, p, *, n_nodes, d, blk):
    def body(x_ref, p_ref, o_ref):
        o_ref[...] = x_ref[...] + p_ref[0] + p_ref[1]

    return pl.pallas_call(
        body,
        grid=(n_nodes // blk,),
        in_specs=[
            pl.BlockSpec((blk, d), lambda i: (i, 0)),
            pl.BlockSpec((NC, blk, d), lambda i: (0, i, 0)),
        ],
        out_specs=pl.BlockSpec((blk, d), lambda i: (i, 0)),
        out_shape=jax.ShapeDtypeStruct((n_nodes, d), jnp.float32),
    )(x, p)


def kernel(x, edge_index):
    n_nodes, d = x.shape
    n_edges = edge_index.shape[1]
    epw = n_edges // NW

    n_pad = ((n_nodes + 8 * NS - 1) // (8 * NS)) * (8 * NS)

    src = edge_index[0].astype(jnp.int32)
    dst = edge_index[1].astype(jnp.int32)

    p = _sc_segment_sum(x, src, dst, n_pad=n_pad, d=d, epw=epw)
    return _combine(x, p, n_nodes=n_nodes, d=d, blk=2000)


# speedup vs baseline: 1.0842x; 1.0842x over previous
import functools

import jax
import jax.numpy as jnp
from jax import lax
from jax.experimental import pallas as pl
from jax.experimental.pallas import tpu as pltpu
from jax.experimental.pallas import tpu_sc as plsc

NC = 2
NS = 16
NW = NC * NS
CHUNK = 128
NB = 3


def _sc_segment_sum(x, src, dst, *, n_pad, d, epw):
    mesh = plsc.VectorSubcoreMesh(core_axis_name="c", subcore_axis_name="s")
    rows_per_sub = n_pad // NS
    n_full = epw // CHUNK
    tail = epw - n_full * CHUNK
    assert n_full % NB == 0 and n_full > 2 * NB
    assert tail % 8 == 0

    @functools.partial(
        pl.kernel,
        out_type=jax.ShapeDtypeStruct((NC, n_pad, d), jnp.float32),
        mesh=mesh,
        scratch_types=(
            [pltpu.VMEM((CHUNK, d), jnp.float32)] * NB
            + [pltpu.VMEM((CHUNK,), jnp.int32)] * NB
            + [pltpu.VMEM((CHUNK,), jnp.int32)] * NB
            + [pltpu.VMEM((max(tail, 8),), jnp.int32)]
            + [pltpu.VMEM_SHARED((n_pad, d), jnp.float32)]
            + [pltpu.SemaphoreType.DMA] * (4 * NB)
        ),
    )
    def k(x_hbm, src_hbm, dst_hbm, out_hbm, *refs):
        buf = refs[0:NB]
        srcb = refs[NB:2 * NB]
        dstb = refs[2 * NB:3 * NB]
        dstt = refs[3 * NB]
        acc = refs[3 * NB + 1]
        sg = refs[3 * NB + 2:4 * NB + 2]
        ss = refs[4 * NB + 2:5 * NB + 2]
        ssrc = refs[5 * NB + 2:6 * NB + 2]
        sd = refs[6 * NB + 2:7 * NB + 2]

        c = lax.axis_index("c")
        s = lax.axis_index("s")
        wid = c * NS + s

        ebase = wid * epw

        def src_stage(m, q):
            pltpu.async_copy(src_hbm.at[pl.ds(ebase + m * CHUNK, CHUNK)],
                             srcb[q], ssrc[q])

        def src_wait(m, q):
            pltpu.make_async_copy(src_hbm.at[pl.ds(ebase + m * CHUNK, CHUNK)],
                                  srcb[q], ssrc[q]).wait()

        def dst_stage(m, q):
            pltpu.async_copy(dst_hbm.at[pl.ds(ebase + m * CHUNK, CHUNK)],
                             dstb[q], sd[q])

        def dst_wait(m, q):
            pltpu.make_async_copy(dst_hbm.at[pl.ds(ebase + m * CHUNK, CHUNK)],
                                  dstb[q], sd[q]).wait()

        def gather(q):
            pltpu.async_copy(x_hbm.at[srcb[q]], buf[q], sg[q])

        def gather_wait(q):
            pltpu.make_async_copy(x_hbm.at[srcb[q]], buf[q], sg[q]).wait()

        def scat(q):
            pltpu.async_copy(buf[q], acc.at[dstb[q]], ss[q], add=True)

        def scat_wait(q):
            pltpu.make_async_copy(buf[q], acc.at[dstb[q]], ss[q]).wait()

        for q in range(NB):
            src_stage(q, q)
        dst_stage(0, 0)
        dst_stage(1, 1)

        zv = jnp.zeros((16,), jnp.float32)
        zb = buf[NB - 1]

        @pl.loop(0, CHUNK)
        def _(r):
            @pl.loop(0, d, step=16)
            def _(q):
                zb.at[r][pl.ds(q, 16)] = zv

        base = s * rows_per_sub
        nfull_z = rows_per_sub // CHUNK
        rem_z = rows_per_sub - nfull_z * CHUNK

        @pl.loop(0, nfull_z)
        def _(t):
            pltpu.sync_copy(zb, acc.at[pl.ds(base + t * CHUNK, CHUNK)])

        if rem_z:
            pltpu.sync_copy(
                zb.at[pl.ds(0, rem_z)],
                acc.at[pl.ds(base + nfull_z * CHUNK, rem_z)],
            )

        src_wait(0, 0)
        gather(0)
        src_wait(1, 1)
        gather(1)
        plsc.subcore_barrier()

        @pl.loop(0, n_full, step=NB)
        def _(j):
            for q in range(NB):
                m = j + q
                gather_wait(q)

                @pl.when(m + NB < n_full)
                def _():
                    src_stage(m + NB, q)

                dst_wait(m, q)
                scat(q)

                @pl.when(m >= 1)
                def _():
                    scat_wait((q + 2) % NB)

                @pl.when(m + 2 < n_full)
                def _():
                    qn = (q + 2) % NB
                    dst_stage(m + 2, qn)
                    src_wait(m + 2, qn)
                    gather(qn)

        scat_wait((n_full - 1) % NB)

        if tail:
            t0 = ebase + n_full * CHUNK
            q = n_full % NB
            pltpu.async_copy(dst_hbm.at[pl.ds(t0, tail)], dstt, sd[q])
            pltpu.async_copy(src_hbm.at[pl.ds(t0, tail)],
                             srcb[q].at[pl.ds(0, tail)], ssrc[q])
            pltpu.make_async_copy(src_hbm.at[pl.ds(t0, tail)],
                                  srcb[q].at[pl.ds(0, tail)], ssrc[q]).wait()
            pltpu.async_copy(
                x_hbm.at[srcb[q].at[pl.ds(0, tail)]],
                buf[q].at[pl.ds(0, tail)], sg[q],
            )
            pltpu.make_async_copy(
                x_hbm.at[srcb[q].at[pl.ds(0, tail)]],
                buf[q].at[pl.ds(0, tail)], sg[q],
            ).wait()
            pltpu.make_async_copy(dst_hbm.at[pl.ds(t0, tail)],
                                  dstt, sd[q]).wait()
            pltpu.sync_copy(buf[q].at[pl.ds(0, tail)], acc.at[dstt], add=True)

        plsc.subcore_barrier()
        pltpu.sync_copy(
            acc.at[pl.ds(base, rows_per_sub)],
            out_hbm.at[c, pl.ds(base, rows_per_sub)],
        )

    return k(x, src, dst)


def _combine(x, p, *, n_nodes, d, blk):
    def body(x_ref, p_ref, o_ref):
        o_ref[...] = x_ref[...] + p_ref[0] + p_ref[1]

    return pl.pallas_call(
        body,
        grid=(n_nodes // blk,),
        in_specs=[
            pl.BlockSpec((blk, d), lambda i: (i, 0)),
            pl.BlockSpec((NC, blk, d), lambda i: (0, i, 0)),
        ],
        out_specs=pl.BlockSpec((blk, d), lambda i: (i, 0)),
        out_shape=jax.ShapeDtypeStruct((n_nodes, d), jnp.float32),
    )(x, p)


def kernel(x, edge_index):
    n_nodes, d = x.shape
    n_edges = edge_index.shape[1]
    epw = n_edges // NW

    n_pad = ((n_nodes + 8 * NS - 1) // (8 * NS)) * (8 * NS)

    src = edge_index[0].astype(jnp.int32)
    dst = edge_index[1].astype(jnp.int32)

    p = _sc_segment_sum(x, src, dst, n_pad=n_pad, d=d, epw=epw)
    return _combine(x, p, n_nodes=n_nodes, d=d, blk=2000)
